# baseline (speedup 1.0000x reference)
import functools

import jax
import jax.numpy as jnp
from jax import lax
from jax.experimental import pallas as pl
from jax.experimental.pallas import tpu as pltpu
from jax.experimental.pallas import tpu_sc as plsc

D_MODEL = 1024
D_FF = 2048
N_EXPERTS = 8
SEQ = 2048
N_PAIRS = 2 * SEQ
TILE = 256
N_TILES = 24
S_SLOTS = N_TILES * TILE

NC = 2
NW = 32


def _cumsum_lanes(m):
    n = m.shape[1]
    s = 1
    while s < n:
        shifted = jnp.concatenate(
            [jnp.zeros((m.shape[0], s), m.dtype), m[:, :-s]], axis=1)
        m = m + shifted
        s *= 2
    return m


def _cumsum_sublanes(m):
    n = m.shape[0]
    s = 1
    while s < n:
        shifted = jnp.concatenate(
            [jnp.zeros((s, m.shape[1]), m.dtype), m[:-s, :]], axis=0)
        m = m + shifted
        s *= 2
    return m


def _router_body(x_ref, gw_ref, pos_ref, wab_ref, meta_ref, xb_ref):
    xb_ref[...] = x_ref[...].astype(jnp.bfloat16)
    lg = jnp.dot(x_ref[...], gw_ref[...].T,
                 preferred_element_type=jnp.float32)
    v0 = jnp.max(lg, axis=1, keepdims=True)
    is0 = (lg == v0).astype(jnp.int32)
    sel0 = (is0 * (_cumsum_lanes(is0) == 1)).astype(jnp.int32)
    neg = jnp.float32(-3.0e38)
    lg1 = jnp.where(sel0 == 1, neg, lg)
    v1 = jnp.max(lg1, axis=1, keepdims=True)
    is1 = (lg1 == v1).astype(jnp.int32)
    sel1 = (is1 * (_cumsum_lanes(is1) == 1)).astype(jnp.int32)

    s = jnp.exp(v1 - v0)
    w_top = 1.0 / (1.0 + s)
    w_sec = s / (1.0 + s)
    wab_ref[...] = jnp.concatenate(
        [jnp.broadcast_to(w_top, (SEQ, 16)),
         jnp.broadcast_to(w_sec, (SEQ, 16))], axis=1)

    onehot = jnp.concatenate([sel0, sel1], axis=0)
    csum = _cumsum_sublanes(onehot)
    rank = jnp.sum(onehot * (csum - 1), axis=1, keepdims=True)
    counts = csum[N_PAIRS - 1:N_PAIRS, :]

    tcnt = (counts + (TILE - 1)) // TILE
    tend = _cumsum_lanes(tcnt)
    tstart = tend - tcnt
    seg = jnp.sum(onehot * (tstart * TILE), axis=1, keepdims=True)
    pos_ref[...] = rank + seg

    e_row = lax.broadcasted_iota(jnp.int32, (128, N_EXPERTS), 1)
    t_col = lax.broadcasted_iota(jnp.int32, (128, N_EXPERTS), 0)
    in_seg = ((t_col >= tstart) & (t_col < tend)).astype(jnp.int32)
    te = jnp.sum(e_row * in_seg, axis=1, keepdims=True)
    act = jnp.sum(in_seg, axis=1, keepdims=True)
    last_e = jnp.max(e_row[:1, :] * (counts > 0), axis=1, keepdims=True)
    te = jnp.where(act > 0, te, last_e)
    meta_ref[...] = jnp.concatenate([te, act], axis=1)[:N_TILES]


def _router(x2, gate_w):
    return pl.pallas_call(
        _router_body,
        in_specs=[
            pl.BlockSpec((SEQ, D_MODEL), lambda: (0, 0)),
            pl.BlockSpec((N_EXPERTS, D_MODEL), lambda: (0, 0)),
        ],
        out_specs=[
            pl.BlockSpec((N_PAIRS, 1), lambda: (0, 0)),
            pl.BlockSpec((SEQ, 32), lambda: (0, 0)),
            pl.BlockSpec((N_TILES, 2), lambda: (0, 0)),
            pl.BlockSpec((SEQ, D_MODEL), lambda: (0, 0)),
        ],
        out_shape=[
            jax.ShapeDtypeStruct((N_PAIRS, 1), jnp.int32),
            jax.ShapeDtypeStruct((SEQ, 32), jnp.float32),
            jax.ShapeDtypeStruct((N_TILES, 2), jnp.int32),
            jax.ShapeDtypeStruct((SEQ, D_MODEL), jnp.bfloat16),
        ],
    )(x2, gate_w)


_CHUNK_B = 64


@functools.cache
def _sc_mesh():
    return plsc.VectorSubcoreMesh(core_axis_name="c", subcore_axis_name="s")


@functools.cache
def _scatter_kernel():
    @functools.partial(
        pl.kernel,
        out_type=jax.ShapeDtypeStruct((S_SLOTS, D_MODEL // 2), jnp.int32),
        mesh=_sc_mesh(),
        scratch_types=[
            pltpu.VMEM((_CHUNK_B,), jnp.int32),
            pltpu.VMEM((_CHUNK_B, D_MODEL // 2), jnp.int32),
        ],
    )
    def _scatter_body(x_hbm, pos_hbm, xs_hbm, idx_v, rows_v):
        wid = lax.axis_index("s") * NC + lax.axis_index("c")
        per_w = N_PAIRS // NW
        for c in range(per_w // _CHUNK_B):
            base = wid * per_w + c * _CHUNK_B
            pltpu.sync_copy(pos_hbm.at[pl.ds(base, _CHUNK_B)], idx_v)
            src = base % SEQ
            pltpu.sync_copy(x_hbm.at[pl.ds(src, _CHUNK_B)], rows_v)
            pltpu.sync_copy(rows_v, xs_hbm.at[idx_v])

    return _scatter_body


def _scatter_k(x2, pos):
    return _scatter_kernel()(x2, pos)


def _ffn_body(s_ref, xs_ref, wg_ref, w1_ref, w2_ref, o_ref):
    i = pl.program_id(0)

    @pl.when(s_ref[i, 1] == 1)
    def _():
        fast = jax.lax.Precision.DEFAULT
        xt = xs_ref[...].astype(jnp.float32)
        g = jnp.dot(xt, wg_ref[0].T, precision=fast,
                    preferred_element_type=jnp.float32)
        a = jnp.dot(xt, w1_ref[0].T, precision=fast,
                    preferred_element_type=jnp.float32)
        h = (g * jax.nn.sigmoid(g)) * a
        o_ref[...] = jnp.dot(h, w2_ref[0].T, precision=fast,
                             preferred_element_type=jnp.float32)


def _ffn(xs, wgb, w1b, w2b, meta):
    grid_spec = pltpu.PrefetchScalarGridSpec(
        num_scalar_prefetch=1,
        grid=(N_TILES,),
        in_specs=[
            pl.BlockSpec((TILE, D_MODEL), lambda i, s: (i, 0)),
            pl.BlockSpec((1, D_FF, D_MODEL), lambda i, s: (s[i, 0], 0, 0)),
            pl.BlockSpec((1, D_FF, D_MODEL), lambda i, s: (s[i, 0], 0, 0)),
            pl.BlockSpec((1, D_MODEL, D_FF), lambda i, s: (s[i, 0], 0, 0)),
        ],
        out_specs=pl.BlockSpec((TILE, D_MODEL), lambda i, s: (i, 0)),
    )
    return pl.pallas_call(
        _ffn_body,
        grid_spec=grid_spec,
        out_shape=jax.ShapeDtypeStruct((S_SLOTS, D_MODEL), jnp.float32),
    )(meta, xs, wgb, w1b, w2b)


_CHUNK_D = 32


@functools.cache
def _combine_kernel():
    @functools.partial(
        pl.kernel,
        out_type=jax.ShapeDtypeStruct((SEQ, D_MODEL), jnp.float32),
        mesh=_sc_mesh(),
        scratch_types=[
            pltpu.VMEM((_CHUNK_D,), jnp.int32),
            pltpu.VMEM((_CHUNK_D,), jnp.int32),
            pltpu.VMEM((SEQ // NW, 32), jnp.float32),
            pltpu.VMEM((_CHUNK_D, D_MODEL), jnp.float32),
            pltpu.VMEM((_CHUNK_D, D_MODEL), jnp.float32),
        ],
    )
    def _combine_body(ys_hbm, pos_hbm, wab_hbm, out_hbm,
                      ia_v, ib_v, wab_v, ra_v, rb_v):
        wid = lax.axis_index("s") * NC + lax.axis_index("c")
        tok0 = wid * (SEQ // NW)
        pltpu.sync_copy(wab_hbm.at[pl.ds(tok0, SEQ // NW)], wab_v)
        for c in range((SEQ // NW) // _CHUNK_D):
            tok = tok0 + c * _CHUNK_D
            pltpu.sync_copy(pos_hbm.at[pl.ds(tok, _CHUNK_D)], ia_v)
            pltpu.sync_copy(pos_hbm.at[pl.ds(SEQ + tok, _CHUNK_D)], ib_v)
            pltpu.sync_copy(ys_hbm.at[ia_v], ra_v)
            pltpu.sync_copy(ys_hbm.at[ib_v], rb_v)

            def row_body(i, _, c=c):
                r = c * _CHUNK_D + i
                wa = wab_v[r, pl.ds(0, 16)]
                wb = wab_v[r, pl.ds(16, 16)]
                for j in range(D_MODEL // 16):
                    a = ra_v[i, pl.ds(j * 16, 16)]
                    b = rb_v[i, pl.ds(j * 16, 16)]
                    ra_v[i, pl.ds(j * 16, 16)] = a * wa + b * wb
                return 0

            lax.fori_loop(0, _CHUNK_D, row_body, 0)
            pltpu.sync_copy(ra_v, out_hbm.at[pl.ds(tok, _CHUNK_D)])

    return _combine_body


def _combine_k(ys, pos, wab):
    return _combine_kernel()(ys, pos, wab)


def kernel(x, gate_w, w1, w_gate, w2):
    B, T, D = x.shape
    x2 = x.reshape(T, D)

    pos2, wab, meta, xb = _router(x2, gate_w)
    pos = pos2.reshape(N_PAIRS)
    xb32 = lax.bitcast_convert_type(
        xb.reshape(SEQ, D_MODEL // 2, 2), jnp.int32)
    xs32 = _scatter_k(xb32, pos)
    xs = lax.bitcast_convert_type(xs32, jnp.bfloat16).reshape(S_SLOTS, D_MODEL)
    ys = _ffn(xs, w_gate, w1, w2, meta)
    out = _combine_k(ys, pos, wab)
    return out.reshape(B, T, D)


# speedup vs baseline: 1.9160x; 1.9160x over previous
import functools

import jax
import jax.numpy as jnp
from jax import lax
from jax.experimental import pallas as pl
from jax.experimental.pallas import tpu as pltpu
from jax.experimental.pallas import tpu_sc as plsc

D_MODEL = 1024
D_FF = 2048
N_EXPERTS = 8
SEQ = 2048
N_PAIRS = 2 * SEQ
TILE = 256
N_TILES = 24
S_SLOTS = N_TILES * TILE

NC = 2
NW = 32


def _cumsum_lanes(m):
    n = m.shape[1]
    s = 1
    while s < n:
        shifted = jnp.concatenate(
            [jnp.zeros((m.shape[0], s), m.dtype), m[:, :-s]], axis=1)
        m = m + shifted
        s *= 2
    return m


def _cumsum_sublanes(m):
    n = m.shape[0]
    s = 1
    while s < n:
        shifted = jnp.concatenate(
            [jnp.zeros((s, m.shape[1]), m.dtype), m[:-s, :]], axis=0)
        m = m + shifted
        s *= 2
    return m


def _router_body(x_ref, gw_ref, pos_ref, wab_ref, meta_ref):
    lg = jnp.dot(x_ref[...], gw_ref[...].T,
                 preferred_element_type=jnp.float32)
    v0 = jnp.max(lg, axis=1, keepdims=True)
    is0 = (lg == v0).astype(jnp.int32)
    sel0 = (is0 * (_cumsum_lanes(is0) == 1)).astype(jnp.int32)
    neg = jnp.float32(-3.0e38)
    lg1 = jnp.where(sel0 == 1, neg, lg)
    v1 = jnp.max(lg1, axis=1, keepdims=True)
    is1 = (lg1 == v1).astype(jnp.int32)
    sel1 = (is1 * (_cumsum_lanes(is1) == 1)).astype(jnp.int32)

    s = jnp.exp(v1 - v0)
    w_top = 1.0 / (1.0 + s)
    w_sec = s / (1.0 + s)
    wab_ref[...] = jnp.concatenate(
        [jnp.broadcast_to(w_top, (SEQ, 16)),
         jnp.broadcast_to(w_sec, (SEQ, 16))], axis=1)

    onehot = jnp.concatenate([sel0, sel1], axis=0)
    csum = _cumsum_sublanes(onehot)
    rank = jnp.sum(onehot * (csum - 1), axis=1, keepdims=True)
    counts = csum[N_PAIRS - 1:N_PAIRS, :]

    tcnt = (counts + (TILE - 1)) // TILE
    tend = _cumsum_lanes(tcnt)
    tstart = tend - tcnt
    seg = jnp.sum(onehot * (tstart * TILE), axis=1, keepdims=True)
    pos_ref[...] = rank + seg

    e_row = lax.broadcasted_iota(jnp.int32, (128, N_EXPERTS), 1)
    t_col = lax.broadcasted_iota(jnp.int32, (128, N_EXPERTS), 0)
    in_seg = ((t_col >= tstart) & (t_col < tend)).astype(jnp.int32)
    te = jnp.sum(e_row * in_seg, axis=1, keepdims=True)
    act = jnp.sum(in_seg, axis=1, keepdims=True)
    last_e = jnp.max(e_row[:1, :] * (counts > 0), axis=1, keepdims=True)
    te = jnp.where(act > 0, te, last_e)
    meta_ref[...] = jnp.concatenate([te, act], axis=1)[:N_TILES]


def _router(x2, gate_w):
    return pl.pallas_call(
        _router_body,
        in_specs=[
            pl.BlockSpec((SEQ, D_MODEL), lambda: (0, 0)),
            pl.BlockSpec((N_EXPERTS, D_MODEL), lambda: (0, 0)),
        ],
        out_specs=[
            pl.BlockSpec((N_PAIRS, 1), lambda: (0, 0)),
            pl.BlockSpec((SEQ, 32), lambda: (0, 0)),
            pl.BlockSpec((N_TILES, 2), lambda: (0, 0)),
        ],
        out_shape=[
            jax.ShapeDtypeStruct((N_PAIRS, 1), jnp.int32),
            jax.ShapeDtypeStruct((SEQ, 32), jnp.float32),
            jax.ShapeDtypeStruct((N_TILES, 2), jnp.int32),
        ],
    )(x2, gate_w)


_CHUNK_B = 64


@functools.cache
def _sc_mesh():
    return plsc.VectorSubcoreMesh(core_axis_name="c", subcore_axis_name="s")


@functools.cache
def _scatter_kernel():
    @functools.partial(
        pl.kernel,
        out_type=jax.ShapeDtypeStruct((S_SLOTS, D_MODEL), jnp.float32),
        mesh=_sc_mesh(),
        scratch_types=[
            pltpu.VMEM((_CHUNK_B,), jnp.int32),
            pltpu.VMEM((_CHUNK_B, D_MODEL), jnp.float32),
        ],
    )
    def _scatter_body(x_hbm, pos_hbm, xs_hbm, idx_v, rows_v):
        wid = lax.axis_index("s") * NC + lax.axis_index("c")
        per_w = N_PAIRS // NW
        for c in range(per_w // _CHUNK_B):
            base = wid * per_w + c * _CHUNK_B
            pltpu.sync_copy(pos_hbm.at[pl.ds(base, _CHUNK_B)], idx_v)
            src = base % SEQ
            pltpu.sync_copy(x_hbm.at[pl.ds(src, _CHUNK_B)], rows_v)
            pltpu.sync_copy(rows_v, xs_hbm.at[idx_v])

    return _scatter_body


def _scatter_k(x2, pos):
    return _scatter_kernel()(x2, pos)


def _ffn_body(s_ref, xs_ref, wg_ref, w1_ref, w2_ref, o_ref):
    i = pl.program_id(0)

    @pl.when(s_ref[i, 1] == 1)
    def _():
        fast = jax.lax.Precision.DEFAULT
        xt = xs_ref[...]
        g = jnp.dot(xt, wg_ref[0].T, precision=fast,
                    preferred_element_type=jnp.float32)
        a = jnp.dot(xt, w1_ref[0].T, precision=fast,
                    preferred_element_type=jnp.float32)
        h = (g * jax.nn.sigmoid(g)) * a
        o_ref[...] = jnp.dot(h, w2_ref[0].T, precision=fast,
                             preferred_element_type=jnp.float32)


def _ffn(xs, wgb, w1b, w2b, meta):
    grid_spec = pltpu.PrefetchScalarGridSpec(
        num_scalar_prefetch=1,
        grid=(N_TILES,),
        in_specs=[
            pl.BlockSpec((TILE, D_MODEL), lambda i, s: (i, 0)),
            pl.BlockSpec((1, D_FF, D_MODEL), lambda i, s: (s[i, 0], 0, 0)),
            pl.BlockSpec((1, D_FF, D_MODEL), lambda i, s: (s[i, 0], 0, 0)),
            pl.BlockSpec((1, D_MODEL, D_FF), lambda i, s: (s[i, 0], 0, 0)),
        ],
        out_specs=pl.BlockSpec((TILE, D_MODEL), lambda i, s: (i, 0)),
    )
    return pl.pallas_call(
        _ffn_body,
        grid_spec=grid_spec,
        out_shape=jax.ShapeDtypeStruct((S_SLOTS, D_MODEL), jnp.float32),
    )(meta, xs, wgb, w1b, w2b)


_CHUNK_D = 32


@functools.cache
def _combine_kernel():
    @functools.partial(
        pl.kernel,
        out_type=jax.ShapeDtypeStruct((SEQ, D_MODEL), jnp.float32),
        mesh=_sc_mesh(),
        scratch_types=[
            pltpu.VMEM((_CHUNK_D,), jnp.int32),
            pltpu.VMEM((_CHUNK_D,), jnp.int32),
            pltpu.VMEM((SEQ // NW, 32), jnp.float32),
            pltpu.VMEM((_CHUNK_D, D_MODEL), jnp.float32),
            pltpu.VMEM((_CHUNK_D, D_MODEL), jnp.float32),
        ],
    )
    def _combine_body(ys_hbm, pos_hbm, wab_hbm, out_hbm,
                      ia_v, ib_v, wab_v, ra_v, rb_v):
        wid = lax.axis_index("s") * NC + lax.axis_index("c")
        tok0 = wid * (SEQ // NW)
        pltpu.sync_copy(wab_hbm.at[pl.ds(tok0, SEQ // NW)], wab_v)
        for c in range((SEQ // NW) // _CHUNK_D):
            tok = tok0 + c * _CHUNK_D
            pltpu.sync_copy(pos_hbm.at[pl.ds(tok, _CHUNK_D)], ia_v)
            pltpu.sync_copy(pos_hbm.at[pl.ds(SEQ + tok, _CHUNK_D)], ib_v)
            pltpu.sync_copy(ys_hbm.at[ia_v], ra_v)
            pltpu.sync_copy(ys_hbm.at[ib_v], rb_v)

            def row_body(i, _, c=c):
                r = c * _CHUNK_D + i
                wa = wab_v[r, pl.ds(0, 16)]
                wb = wab_v[r, pl.ds(16, 16)]
                for j in range(D_MODEL // 16):
                    a = ra_v[i, pl.ds(j * 16, 16)]
                    b = rb_v[i, pl.ds(j * 16, 16)]
                    ra_v[i, pl.ds(j * 16, 16)] = a * wa + b * wb
                return 0

            lax.fori_loop(0, _CHUNK_D, row_body, 0)
            pltpu.sync_copy(ra_v, out_hbm.at[pl.ds(tok, _CHUNK_D)])

    return _combine_body


def _combine_k(ys, pos, wab):
    return _combine_kernel()(ys, pos, wab)


def kernel(x, gate_w, w1, w_gate, w2):
    B, T, D = x.shape
    x2 = x.reshape(T, D)

    pos2, wab, meta = _router(x2, gate_w)
    pos = pos2.reshape(N_PAIRS)
    xs = _scatter_k(x2, pos)
    ys = _ffn(xs, w_gate, w1, w2, meta)
    out = _combine_k(ys, pos, wab)
    return out.reshape(B, T, D)
